# baseline (speedup 1.0000x reference)
import functools

import jax
import jax.numpy as jnp
from jax import lax
from jax.experimental import pallas as pl
from jax.experimental.pallas import tpu as pltpu
from jax.experimental.pallas import tpu_sc as plsc

NUM_FIELDS = 26
FIELD_SIZE = 100000
EMBED_DIM = 64
BATCH = 16384

_info = plsc.get_sparse_core_info()
NC, NS, L = _info.num_cores, _info.num_subcores, _info.num_lanes
NW = NC * NS

BF = BATCH * NUM_FIELDS
B_PER_W = BF // NW
CHUNK = 512
N_CHUNK = B_PER_W // CHUNK
IDX_ROWS = CHUNK // 128


def _tec_body(cat_hbm, w_hbm, out_hbm, idx_j, rows, sem):
    wid = lax.axis_index("s") * NC + lax.axis_index("c")
    base_w = wid * B_PER_W
    iota = lax.iota(jnp.int32, L)

    def chunk_body(ci, _):
        base_g = base_w + ci * CHUNK
        row0 = pl.multiple_of(base_g // 128, IDX_ROWS)
        pltpu.sync_copy(cat_hbm.at[pl.ds(row0, IDX_ROWS)], idx_j)
        for r in range(IDX_ROWS):
            def lane_body(m, _, r=r):
                o = pl.multiple_of(m * L, L)
                g = base_g + r * 128 + m * L
                off = ((g + iota) % NUM_FIELDS) * FIELD_SIZE
                idx_j[r, pl.ds(o, L)] = idx_j[r, pl.ds(o, L)] + off
                return 0
            lax.fori_loop(0, 128 // L, lane_body, 0)
        for r in range(IDX_ROWS):
            pltpu.async_copy(w_hbm.at[idx_j.at[r]],
                             rows.at[pl.ds(r * 128, 128)], sem)
        for r in range(IDX_ROWS):
            pltpu.make_async_copy(w_hbm.at[idx_j.at[r]],
                                  rows.at[pl.ds(r * 128, 128)], sem).wait()
        pltpu.sync_copy(rows,
                        out_hbm.at[pl.ds(pl.multiple_of(base_g, 512), CHUNK)])
        return 0

    lax.fori_loop(0, N_CHUNK, chunk_body, 0)


@jax.jit
def kernel(categorical_inputs, weights):
    cat_flat = categorical_inputs.reshape(BF // 128, 128)
    mesh = plsc.VectorSubcoreMesh(core_axis_name="c", subcore_axis_name="s")
    out_flat = pl.kernel(
        _tec_body,
        out_type=jax.ShapeDtypeStruct((BF, EMBED_DIM), jnp.float32),
        mesh=mesh,
        scratch_types=[
            pltpu.VMEM((IDX_ROWS, 128), jnp.int32),
            pltpu.VMEM((CHUNK, EMBED_DIM), jnp.float32),
            pltpu.SemaphoreType.DMA,
        ],
        compiler_params=pltpu.CompilerParams(use_tc_tiling_on_sc=False),
    )(cat_flat, weights)
    return out_flat.reshape(BATCH, NUM_FIELDS, EMBED_DIM)


# speedup vs baseline: 1.0146x; 1.0146x over previous
import jax
import jax.numpy as jnp
from jax import lax
from jax.experimental import pallas as pl
from jax.experimental.pallas import tpu as pltpu
from jax.experimental.pallas import tpu_sc as plsc

NUM_FIELDS = 26
FIELD_SIZE = 100000
EMBED_DIM = 64
BATCH = 16384

_info = plsc.get_sparse_core_info()
NC, NS, L = _info.num_cores, _info.num_subcores, _info.num_lanes
NW = NC * NS

BF = BATCH * NUM_FIELDS
B_PER_W = BF // NW
SLOT = 128
GROUP_SLOTS = 4
GROUP = SLOT * GROUP_SLOTS
NG = B_PER_W // GROUP


def _tec_body(cat_hbm, w_hbm, out_hbm, jraw, jidx, rows,
              gsem0, gsem1, osem0, osem1, rsem0, rsem1):
    wid = lax.axis_index("s") * NC + lax.axis_index("c")
    base_w = wid * B_PER_W
    iota = lax.iota(jnp.int32, L)
    gsem = (gsem0, gsem1)
    osem = (osem0, osem1)
    rsem = (rsem0, rsem1)

    def raw_start(g, h):
        pltpu.async_copy(cat_hbm.at[wid, g], jraw.at[h], rsem[h])

    def raw_wait(g, h):
        pltpu.make_async_copy(cat_hbm.at[wid, g], jraw.at[h], rsem[h]).wait()

    def compute_jidx(g, h):
        for r in range(GROUP_SLOTS):
            def lane(m, _, r=r):
                o = pl.multiple_of(m * L, L)
                pos = base_w + g * GROUP + r * SLOT + o
                off = ((pos + iota) % NUM_FIELDS) * FIELD_SIZE
                jidx[h, r, pl.ds(o, L)] = jraw[h, r, pl.ds(o, L)] + off
                return 0
            lax.fori_loop(0, SLOT // L, lane, 0)

    def gather_start(h):
        for r in range(GROUP_SLOTS):
            pltpu.async_copy(w_hbm.at[jidx.at[h, r]],
                             rows.at[h, pl.ds(r * SLOT, SLOT)], gsem[h])

    def gather_wait(h):
        for r in range(GROUP_SLOTS):
            pltpu.make_async_copy(w_hbm.at[jidx.at[h, r]],
                                  rows.at[h, pl.ds(r * SLOT, SLOT)],
                                  gsem[h]).wait()

    def out_start(g, h):
        obase = pl.multiple_of(base_w + g * GROUP, GROUP)
        pltpu.async_copy(rows.at[h], out_hbm.at[pl.ds(obase, GROUP)], osem[h])

    def out_wait(g, h):
        obase = pl.multiple_of(base_w + g * GROUP, GROUP)
        pltpu.make_async_copy(rows.at[h], out_hbm.at[pl.ds(obase, GROUP)],
                              osem[h]).wait()

    raw_start(0, 0)
    raw_wait(0, 0)
    compute_jidx(0, 0)
    gather_start(0)
    raw_start(1, 1)

    def body(g2, _):
        for h in range(2):
            g = g2 * 2 + h
            h2 = 1 - h

            @pl.when(g + 1 < NG)
            def _():
                raw_wait(g + 1, h2)
                compute_jidx(g + 1, h2)

            gather_wait(h)

            @pl.when(g >= 1)
            def _():
                out_wait(g - 1, h2)

            @pl.when(g + 1 < NG)
            def _():
                gather_start(h2)

            out_start(g, h)

            @pl.when(g + 2 < NG)
            def _():
                raw_start(g + 2, h)
        return 0

    lax.fori_loop(0, NG // 2, body, 0)
    out_wait(NG - 1, 1)


@jax.jit
def kernel(categorical_inputs, weights):
    cat4d = categorical_inputs.reshape(NW, NG, GROUP_SLOTS, SLOT)
    mesh = plsc.VectorSubcoreMesh(core_axis_name="c", subcore_axis_name="s")
    out_flat = pl.kernel(
        _tec_body,
        out_type=jax.ShapeDtypeStruct((BF, EMBED_DIM), jnp.float32),
        mesh=mesh,
        scratch_types=[
            pltpu.VMEM((2, GROUP_SLOTS, SLOT), jnp.int32),
            pltpu.VMEM((2, GROUP_SLOTS, SLOT), jnp.int32),
            pltpu.VMEM((2, GROUP, EMBED_DIM), jnp.float32),
            pltpu.SemaphoreType.DMA,
            pltpu.SemaphoreType.DMA,
            pltpu.SemaphoreType.DMA,
            pltpu.SemaphoreType.DMA,
            pltpu.SemaphoreType.DMA,
            pltpu.SemaphoreType.DMA,
        ],
        compiler_params=pltpu.CompilerParams(use_tc_tiling_on_sc=False),
    )(cat4d, weights)
    return out_flat.reshape(BATCH, NUM_FIELDS, EMBED_DIM)
